# baseline (speedup 1.0000x reference)
import functools

import jax
import jax.numpy as jnp
from jax import lax
from jax.experimental import pallas as pl
from jax.experimental.pallas import tpu as pltpu
from jax.experimental.pallas import tpu_sc as plsc

B, S, V, D, C = 4096, 200, 100000, 100, 90
NC, NS = 2, 16
NW = NC * NS
BPW = B // NW
HALF = S // 2
DPAD = 128
NCH = DPAD // 16


OFFS = (0, 16, 32, 48, 64, 80, 84)
UNROLL = 4


def _pool_body(ids_hbm, table_hbm, out_hbm, idx_v, g_v, out_v, sem0, sem1):
    w = lax.axis_index("s") * NC + lax.axis_index("c")
    pltpu.sync_copy(ids_hbm.at[pl.ds(w * 2 * BPW, 2 * BPW)], idx_v)
    zero = jnp.zeros((16,), jnp.float32)
    sems = (sem0, sem1)

    def zero_pad(b, _):
        out_v[b, pl.ds(96, 16)] = zero
        out_v[b, pl.ds(112, 16)] = zero
        return 0

    lax.fori_loop(0, BPW, zero_pad, 0)

    def issue(b, buf, sem):
        pltpu.async_copy(
            table_hbm.at[idx_v.at[2 * b]], g_v.at[buf, pl.ds(0, HALF)], sem)
        pltpu.async_copy(
            table_hbm.at[idx_v.at[2 * b + 1]],
            g_v.at[buf, pl.ds(HALF, HALF)], sem)

    def drain(b, buf, sem):
        pltpu.make_async_copy(
            table_hbm.at[idx_v.at[2 * b]],
            g_v.at[buf, pl.ds(0, HALF)], sem).wait()
        pltpu.make_async_copy(
            table_hbm.at[idx_v.at[2 * b + 1]],
            g_v.at[buf, pl.ds(HALF, HALF)], sem).wait()

    def fold(b, buf):
        def tok_step(rq, acc):
            for u in range(UNROLL):
                r = UNROLL * rq + u
                acc = tuple(
                    acc[j] + g_v[buf, r, pl.ds(OFFS[j], 16)]
                    for j in range(len(OFFS)))
            return acc

        acc = lax.fori_loop(0, S // UNROLL, tok_step, (zero,) * len(OFFS))
        for j in range(len(OFFS)):
            out_v[b, pl.ds(OFFS[j], 16)] = acc[j]

    issue(0, 0, sems[0])

    def pair_step(bp, _):
        b0 = 2 * bp
        b1 = b0 + 1
        issue(b1, 1, sems[1])
        drain(b0, 0, sems[0])
        fold(b0, 0)

        @pl.when(bp < BPW // 2 - 1)
        def _():
            issue(b0 + 2, 0, sems[0])

        drain(b1, 1, sems[1])
        fold(b1, 1)
        return 0

    lax.fori_loop(0, BPW // 2, pair_step, 0)
    pltpu.sync_copy(out_v, out_hbm.at[pl.ds(w * BPW, BPW)])


_pool = functools.partial(
    pl.kernel,
    out_type=jax.ShapeDtypeStruct((B, DPAD), jnp.float32),
    mesh=plsc.VectorSubcoreMesh(core_axis_name="c", subcore_axis_name="s"),
    scratch_types=[
        pltpu.VMEM((2 * BPW, HALF), jnp.int32),
        pltpu.VMEM((2, S, DPAD), jnp.float32),
        pltpu.VMEM((BPW, DPAD), jnp.float32),
        pltpu.SemaphoreType.DMA,
        pltpu.SemaphoreType.DMA,
    ],
)(_pool_body)


_RBLK = 25000


def _pad_table(embedding):
    return jnp.concatenate(
        [embedding, jnp.zeros((V, DPAD - D), jnp.float32)], axis=1)


def _mm_body(p_ref, w_ref, b_ref, o_ref):
    o_ref[...] = (
        jnp.dot(p_ref[...], w_ref[...], preferred_element_type=jnp.float32)
        * (1.0 / S) + b_ref[...])


def kernel(input_ids, embedding, fc_w, fc_b):
    ids2 = input_ids.reshape(2 * B, HALF)
    table_pad = _pad_table(embedding)
    sums = _pool(ids2, table_pad)
    fc_w_pad = jnp.pad(fc_w, ((0, DPAD - D), (0, 0)))
    gb = 512
    return pl.pallas_call(
        _mm_body,
        grid=(B // gb,),
        in_specs=[
            pl.BlockSpec((gb, DPAD), lambda i: (i, 0)),
            pl.BlockSpec((DPAD, C), lambda i: (0, 0)),
            pl.BlockSpec((1, C), lambda i: (0, 0)),
        ],
        out_specs=pl.BlockSpec((gb, C), lambda i: (i, 0)),
        out_shape=jax.ShapeDtypeStruct((B, C), jnp.float32),
    )(sums, fc_w_pad, fc_b.reshape(1, C))


# speedup vs baseline: 1.6696x; 1.6696x over previous
import functools

import jax
import jax.numpy as jnp
from jax import lax
from jax.experimental import pallas as pl
from jax.experimental.pallas import tpu as pltpu
from jax.experimental.pallas import tpu_sc as plsc

B, S, V, D, C = 4096, 200, 100000, 100, 90
NC, NS = 2, 16
NW = NC * NS
BPW = B // NW
HALF = S // 2
DPAD = 128
NCH = DPAD // 16


OFFS = (0, 16, 32, 48, 64, 80, 84)
UNROLL = 4


def _pool_body(ids_hbm, table_hbm, out_hbm, idx_v, g_v, out_v, sem0, sem1):
    w = lax.axis_index("s") * NC + lax.axis_index("c")
    pltpu.sync_copy(ids_hbm.at[pl.ds(w * 2 * BPW, 2 * BPW)], idx_v)
    zero = jnp.zeros((16,), jnp.float32)
    sems = (sem0, sem1)

    def zero_pad(b, _):
        out_v[b, pl.ds(96, 16)] = zero
        out_v[b, pl.ds(112, 16)] = zero
        return 0

    lax.fori_loop(0, BPW, zero_pad, 0)

    def issue(b, buf, sem):
        pltpu.async_copy(
            table_hbm.at[idx_v.at[2 * b]], g_v.at[buf, pl.ds(0, HALF)], sem)
        pltpu.async_copy(
            table_hbm.at[idx_v.at[2 * b + 1]],
            g_v.at[buf, pl.ds(HALF, HALF)], sem)

    def drain(b, buf, sem):
        pltpu.make_async_copy(
            table_hbm.at[idx_v.at[2 * b]],
            g_v.at[buf, pl.ds(0, HALF)], sem).wait()
        pltpu.make_async_copy(
            table_hbm.at[idx_v.at[2 * b + 1]],
            g_v.at[buf, pl.ds(HALF, HALF)], sem).wait()

    def fold(b, buf):
        def tok_step(rq, acc):
            for u in range(UNROLL):
                r = UNROLL * rq + u
                acc = tuple(
                    acc[j] + g_v[buf, r, pl.ds(OFFS[j], 16)]
                    for j in range(len(OFFS)))
            return acc

        acc = lax.fori_loop(0, S // UNROLL, tok_step, (zero,) * len(OFFS))
        for j in range(len(OFFS)):
            out_v[b, pl.ds(OFFS[j], 16)] = acc[j]

    issue(0, 0, sems[0])

    def pair_step(bp, _):
        b0 = 2 * bp
        b1 = b0 + 1
        issue(b1, 1, sems[1])
        drain(b0, 0, sems[0])
        fold(b0, 0)

        @pl.when(bp < BPW // 2 - 1)
        def _():
            issue(b0 + 2, 0, sems[0])

        drain(b1, 1, sems[1])
        fold(b1, 1)
        return 0

    lax.fori_loop(0, BPW // 2, pair_step, 0)
    pltpu.sync_copy(out_v, out_hbm.at[pl.ds(w * BPW, BPW)])


_pool = functools.partial(
    pl.kernel,
    out_type=jax.ShapeDtypeStruct((B, DPAD), jnp.float32),
    mesh=plsc.VectorSubcoreMesh(core_axis_name="c", subcore_axis_name="s"),
    scratch_types=[
        pltpu.VMEM((2 * BPW, HALF), jnp.int32),
        pltpu.VMEM((2, S, DPAD), jnp.float32),
        pltpu.VMEM((BPW, DPAD), jnp.float32),
        pltpu.SemaphoreType.DMA,
        pltpu.SemaphoreType.DMA,
    ],
)(_pool_body)


_RBLK = 12800


def _pad_body(xt_ref, o_ref):
    xt = jnp.swapaxes(xt_ref[...], 0, 1)
    o_ref[...] = jnp.concatenate(
        [xt, jnp.zeros((_RBLK, DPAD - D), jnp.float32)], axis=-1)


def _pad_table(embedding_t):
    grid = (V + _RBLK - 1) // _RBLK
    return pl.pallas_call(
        _pad_body,
        grid=(grid,),
        in_specs=[pl.BlockSpec((D, _RBLK), lambda i: (0, i))],
        out_specs=pl.BlockSpec((_RBLK, DPAD), lambda i: (i, 0)),
        out_shape=jax.ShapeDtypeStruct((V, DPAD), jnp.float32),
    )(embedding_t)


def _mm_body(p_ref, w_ref, b_ref, o_ref):
    o_ref[...] = (
        jnp.dot(p_ref[...], w_ref[...], preferred_element_type=jnp.float32)
        * (1.0 / S) + b_ref[...])


def kernel(input_ids, embedding, fc_w, fc_b):
    ids2 = input_ids.reshape(2 * B, HALF)
    table_pad = _pad_table(embedding.T)
    sums = _pool(ids2, table_pad)
    fc_w_pad = jnp.pad(fc_w, ((0, DPAD - D), (0, 0)))
    gb = 512
    return pl.pallas_call(
        _mm_body,
        grid=(B // gb,),
        in_specs=[
            pl.BlockSpec((gb, DPAD), lambda i: (i, 0)),
            pl.BlockSpec((DPAD, C), lambda i: (0, 0)),
            pl.BlockSpec((1, C), lambda i: (0, 0)),
        ],
        out_specs=pl.BlockSpec((gb, C), lambda i: (i, 0)),
        out_shape=jax.ShapeDtypeStruct((B, C), jnp.float32),
    )(sums, fc_w_pad, fc_b.reshape(1, C))


# speedup vs baseline: 1.9534x; 1.1700x over previous
import functools

import jax
import jax.numpy as jnp
from jax import lax
from jax.experimental import pallas as pl
from jax.experimental.pallas import tpu as pltpu
from jax.experimental.pallas import tpu_sc as plsc

B, S, V, D, C = 4096, 200, 100000, 100, 90
NC, NS = 2, 16
NW = NC * NS
BPW = B // NW
HALF = S // 2
DPAD = 128
NCH = DPAD // 16


OFFS = (0, 16, 32, 48, 64, 80, 84)
UNROLL = 4


def _pool_body(ids_hbm, table_hbm, out_hbm, idx_v, g_v, out_v,
               sem0, sem1, sem2):
    w = lax.axis_index("s") * NC + lax.axis_index("c")
    pltpu.sync_copy(ids_hbm.at[pl.ds(w * 2 * BPW, 2 * BPW)], idx_v)
    zero = jnp.zeros((16,), jnp.float32)
    sems = (sem0, sem1, sem2)

    def zero_pad(b, _):
        out_v[b, pl.ds(96, 16)] = zero
        out_v[b, pl.ds(112, 16)] = zero
        return 0

    lax.fori_loop(0, BPW, zero_pad, 0)

    def issue(b, buf, sem):
        pltpu.async_copy(
            table_hbm.at[idx_v.at[2 * b]], g_v.at[buf, pl.ds(0, HALF)], sem)
        pltpu.async_copy(
            table_hbm.at[idx_v.at[2 * b + 1]],
            g_v.at[buf, pl.ds(HALF, HALF)], sem)

    def drain(b, buf, sem):
        pltpu.make_async_copy(
            table_hbm.at[idx_v.at[2 * b]],
            g_v.at[buf, pl.ds(0, HALF)], sem).wait()
        pltpu.make_async_copy(
            table_hbm.at[idx_v.at[2 * b + 1]],
            g_v.at[buf, pl.ds(HALF, HALF)], sem).wait()

    def fold(b, buf):
        def tok_step(rq, acc):
            for u in range(UNROLL):
                r = UNROLL * rq + u
                acc = tuple(
                    acc[j] + g_v[buf, r, pl.ds(OFFS[j], 16)]
                    for j in range(len(OFFS)))
            return acc

        acc = lax.fori_loop(0, S // UNROLL, tok_step, (zero,) * len(OFFS))
        for j in range(len(OFFS)):
            out_v[b, pl.ds(OFFS[j], 16)] = acc[j]

    issue(0, 0, sems[0])
    issue(1, 1, sems[1])

    def tri_step(t, _):
        b = 3 * t
        issue(b + 2, 2, sems[2])
        drain(b, 0, sems[0])
        fold(b, 0)

        @pl.when(b + 3 < BPW)
        def _():
            issue(b + 3, 0, sems[0])

        drain(b + 1, 1, sems[1])
        fold(b + 1, 1)

        @pl.when(b + 4 < BPW)
        def _():
            issue(b + 4, 1, sems[1])

        drain(b + 2, 2, sems[2])
        fold(b + 2, 2)
        return 0

    lax.fori_loop(0, BPW // 3, tri_step, 0)
    drain(BPW - 2, 0, sems[0])
    fold(BPW - 2, 0)
    drain(BPW - 1, 1, sems[1])
    fold(BPW - 1, 1)
    pltpu.sync_copy(out_v, out_hbm.at[pl.ds(w * BPW, BPW)])


_pool = functools.partial(
    pl.kernel,
    out_type=jax.ShapeDtypeStruct((B, DPAD), jnp.float32),
    mesh=plsc.VectorSubcoreMesh(core_axis_name="c", subcore_axis_name="s"),
    scratch_types=[
        pltpu.VMEM((2 * BPW, HALF), jnp.int32),
        pltpu.VMEM((3, S, DPAD), jnp.float32),
        pltpu.VMEM((BPW, DPAD), jnp.float32),
        pltpu.SemaphoreType.DMA,
        pltpu.SemaphoreType.DMA,
        pltpu.SemaphoreType.DMA,
    ],
)(_pool_body)


_RBLK = 12800


def _pad_body(xt_ref, o_ref):
    xt = jnp.swapaxes(xt_ref[...], 0, 1)
    o_ref[...] = jnp.concatenate(
        [xt, jnp.zeros((_RBLK, DPAD - D), jnp.float32)], axis=-1)


def _pad_table(embedding_t):
    grid = (V + _RBLK - 1) // _RBLK
    return pl.pallas_call(
        _pad_body,
        grid=(grid,),
        in_specs=[pl.BlockSpec((D, _RBLK), lambda i: (0, i))],
        out_specs=pl.BlockSpec((_RBLK, DPAD), lambda i: (i, 0)),
        out_shape=jax.ShapeDtypeStruct((V, DPAD), jnp.float32),
    )(embedding_t)


def _mm_body(p_ref, w_ref, b_ref, o_ref):
    o_ref[...] = (
        jnp.dot(p_ref[...], w_ref[...], preferred_element_type=jnp.float32)
        * (1.0 / S) + b_ref[...])


def kernel(input_ids, embedding, fc_w, fc_b):
    ids2 = input_ids.reshape(2 * B, HALF)
    table_pad = _pad_table(embedding.T)
    sums = _pool(ids2, table_pad)
    fc_w_pad = jnp.pad(fc_w, ((0, DPAD - D), (0, 0)))
    gb = 512
    return pl.pallas_call(
        _mm_body,
        grid=(B // gb,),
        in_specs=[
            pl.BlockSpec((gb, DPAD), lambda i: (i, 0)),
            pl.BlockSpec((DPAD, C), lambda i: (0, 0)),
            pl.BlockSpec((1, C), lambda i: (0, 0)),
        ],
        out_specs=pl.BlockSpec((gb, C), lambda i: (i, 0)),
        out_shape=jax.ShapeDtypeStruct((B, C), jnp.float32),
    )(sums, fc_w_pad, fc_b.reshape(1, C))
